# Initial kernel scaffold; baseline (speedup 1.0000x reference)
import jax
import jax.numpy as jnp
from jax.experimental import pallas as pl


def kernel(data, emb_table, pos_table, f0_W1, f0_b1, f0_W2, f0_b2, f1_W1, f1_b1, f1_W2, f1_b2, f2_W1, f2_b1, f2_W2, f2_b2, g_W1, g_b1, g_W2, g_b2, final_W, final_b, rows, cols):
    raise NotImplementedError("write your pallas kernel here")


# speedup vs baseline: 57.5017x; 57.5017x over previous
import functools

import jax
import jax.numpy as jnp
from jax import lax
from jax.experimental import pallas as pl
from jax.experimental.pallas import tpu as pltpu
from jax.experimental.pallas import tpu_sc as plsc

_B = 4
_N_VEC = 4096
_EMB = 64
_CV = 32
_N_CLASS = 10
_TOK = _B * _N_VEC


def _sc_gather(idx, table):
    info = plsc.get_sparse_core_info()
    _NC, _NS = info.num_cores, info.num_subcores
    _PER_W = _TOK // (_NC * _NS)
    mesh = plsc.VectorSubcoreMesh(core_axis_name="c", subcore_axis_name="s")

    @functools.partial(
        pl.kernel,
        mesh=mesh,
        out_type=jax.ShapeDtypeStruct((_TOK, _EMB), jnp.float32),
        scratch_types=[
            pltpu.VMEM((_PER_W,), jnp.int32),
            pltpu.VMEM((_PER_W, _EMB), jnp.float32),
            pltpu.SemaphoreType.DMA,
        ],
        compiler_params=pltpu.CompilerParams(use_tc_tiling_on_sc=False),
    )
    def gather_kernel(idx_hbm, table_hbm, out_hbm, idx_v, rows_v, sem):
        wid = lax.axis_index("s") * _NC + lax.axis_index("c")
        base = wid * _PER_W
        pltpu.sync_copy(idx_hbm.at[pl.ds(base, _PER_W)], idx_v)
        pltpu.async_copy(table_hbm.at[idx_v], rows_v, sem).wait()
        pltpu.sync_copy(rows_v, out_hbm.at[pl.ds(base, _PER_W)])

    return gather_kernel(idx, table)


def _gelu(u):
    return 0.5 * u * (1.0 + lax.erf(u * 0.7071067811865476))


def _tc_body(x_ref, pos_ref,
             gw1_ref, gb1_ref, gw2_ref, gb2_ref,
             f0w1_ref, f0b1_ref, f0w2_ref, f0b2_ref,
             f1w1_ref, f1b1_ref, f1w2_ref, f1b2_ref,
             f2w1_ref, f2b1_ref, f2w2_ref, f2b2_ref,
             fk_ref, fb_ref, out_ref):
    b = pl.program_id(0)
    x = x_ref[:] + pos_ref[:]

    h = _gelu(jnp.dot(x, gw1_ref[:], preferred_element_type=jnp.float32)
              + gb1_ref[:])
    V = jnp.dot(h, gw2_ref[:], preferred_element_type=jnp.float32) + gb2_ref[:]
    res = V

    Ws = []
    for w1_ref, b1_ref, w2_ref, b2_ref in (
            (f0w1_ref, f0b1_ref, f0w2_ref, f0b2_ref),
            (f1w1_ref, f1b1_ref, f1w2_ref, f1b2_ref),
            (f2w1_ref, f2b1_ref, f2w2_ref, f2b2_ref)):
        hm = _gelu(jnp.dot(x, w1_ref[:], preferred_element_type=jnp.float32)
                   + b1_ref[:])
        Ws.append(jnp.dot(hm, w2_ref[:], preferred_element_type=jnp.float32)
                  + b2_ref[:])

    for m in range(3):
        Wm = Ws[m]
        acc = Wm[:, 0:1] * V
        for j, s in enumerate((1, 2, 4)):
            acc = acc + Wm[:, j + 1:j + 2] * pltpu.roll(V, _N_VEC - s, 0)
        V = acc + res

    prows = [jnp.sum(V * fk_ref[c], axis=0, keepdims=True)
             for c in range(_N_CLASS)]
    P = jnp.concatenate(prows, axis=0)
    col = jnp.sum(P, axis=1, keepdims=True) + fb_ref[:]
    ci = lax.broadcasted_iota(jnp.int32, (_N_CLASS, _B), 1)
    out_ref[:, :] = jnp.where(ci == b,
                              jnp.broadcast_to(col, (_N_CLASS, _B)),
                              out_ref[:, :])


def _tc_call_kwargs():
    full = lambda shape: pl.BlockSpec(shape, lambda b: (0,) * len(shape))
    in_specs = [
        pl.BlockSpec((_N_VEC, _EMB), lambda b: (b, 0)),
        full((_N_VEC, _EMB)),
        full((_EMB, _EMB)), full((1, _EMB)),
        full((_EMB, _CV)), full((1, _CV)),
        full((_EMB, _EMB)), full((1, _EMB)),
        full((_EMB, 4)), full((1, 4)),
        full((_EMB, _EMB)), full((1, _EMB)),
        full((_EMB, 4)), full((1, 4)),
        full((_EMB, _EMB)), full((1, _EMB)),
        full((_EMB, 4)), full((1, 4)),
        full((_N_CLASS, _N_VEC, _CV)),
        full((_N_CLASS, 1)),
    ]
    return dict(
        grid=(_B,),
        in_specs=in_specs,
        out_specs=pl.BlockSpec((_N_CLASS, _B), lambda b: (0, 0)),
        out_shape=jax.ShapeDtypeStruct((_N_CLASS, _B), jnp.float32),
    )


def _tc_forward(x, pos_table, dense_args, fk, fb):
    out_t = pl.pallas_call(_tc_body, **_tc_call_kwargs())(
        x, pos_table, *dense_args, fk, fb)
    return out_t.T


def kernel(data, emb_table, pos_table, f0_W1, f0_b1, f0_W2, f0_b2,
           f1_W1, f1_b1, f1_W2, f1_b2, f2_W1, f2_b1, f2_W2, f2_b2,
           g_W1, g_b1, g_W2, g_b2, final_W, final_b, rows, cols):
    idx = data[..., 0].reshape(_TOK).astype(jnp.int32)
    x = _sc_gather(idx, emb_table)
    fk = final_W.reshape(_N_VEC, _CV, _N_CLASS).transpose(2, 0, 1)
    dense_args = (
        g_W1, g_b1.reshape(1, -1), g_W2, g_b2.reshape(1, -1),
        f0_W1, f0_b1.reshape(1, -1), f0_W2, f0_b2.reshape(1, -1),
        f1_W1, f1_b1.reshape(1, -1), f1_W2, f1_b2.reshape(1, -1),
        f2_W1, f2_b1.reshape(1, -1), f2_W2, f2_b2.reshape(1, -1),
    )
    return _tc_forward(x, pos_table, dense_args, fk, final_b.reshape(-1, 1))
